# Initial kernel scaffold; baseline (speedup 1.0000x reference)
import jax
import jax.numpy as jnp
from jax.experimental import pallas as pl


def kernel(nuclei, params):
    raise NotImplementedError("write your pallas kernel here")


# speedup vs baseline: 222.8778x; 222.8778x over previous
import functools

import numpy as np
import jax
import jax.numpy as jnp
from jax import lax
from jax.experimental import pallas as pl

_N = 24
_CHARGES = np.array([6, 1, 1, 1, 6, 8, 7, 1, 6, 6, 1, 8, 7, 6, 1, 1, 6, 6, 8, 1, 7, 6, 1, 1],
                    dtype=np.int32)
_EMB = 128
_OUT_EMB = 256
_INT_EMB = 64
_NSPH = 7
_NRAD = 6
_NRBF = 32
_CUTOFF = 10.0
_GAIN = 1.6765

_IDX_I, _IDX_J = np.where(np.ones((_N, _N)) - np.eye(_N))
_E = _IDX_I.size

_CM = np.zeros((7, 7))
_CM[0, 0] = 1.0
_CM[1, 1] = 1.0
_CM[2, [0, 2]] = [-0.5, 1.5]
_CM[3, [1, 3]] = [-1.5, 2.5]
_CM[4, [0, 2, 4]] = [3 / 8, -30 / 8, 35 / 8]
_CM[5, [1, 3, 5]] = [15 / 8, -70 / 8, 63 / 8]
_CM[6, [0, 2, 4, 6]] = [-5 / 16, 105 / 16, -315 / 16, 231 / 16]
_CMT = np.repeat(_CM, _NRAD, axis=0).astype(np.float32)


def _onehot(idx, n):
    m = np.zeros((idx.size, n), np.float32)
    m[np.arange(idx.size), idx] = 1.0
    return m


_GI = _onehot(_IDX_I, _N)
_GJ = _onehot(_IDX_J, _N)
_GIT = _GI.T.copy()
_CH = _onehot(_CHARGES, 95)


def _act(x):
    return x * lax.logistic(x) * _GAIN


def _mm(a, b):
    return jnp.dot(a, b, preferred_element_type=jnp.float32)


def _dense(p, x):
    y = _mm(x, p["W"])
    if "b" in p:
        y = y + p["b"]
    return y


def _body(*refs, treedef, n_flat):
    nuc_ref, gi_ref, gj_ref, git_ref, ch_ref, cmt_ref = refs[:6]
    param_refs = refs[6:6 + n_flat]
    out_ref = refs[6 + n_flat]

    p = jax.tree_util.tree_unflatten(treedef, [r[...] for r in param_refs])
    nuc = nuc_ref[...]
    gi = gi_ref[...]
    gj = gj_ref[...]
    git = git_ref[...]
    ch = ch_ref[...]
    cmt = cmt_ref[...]

    diffs = _mm(gi, nuc) - _mm(gj, nuc)
    d2 = jnp.sum(diffs * diffs, axis=1, keepdims=True)
    dinv = lax.rsqrt(jnp.maximum(d2, 1e-24))
    dist = d2 * dinv

    freq = (lax.broadcasted_iota(jnp.int32, (1, _NRBF), 1).astype(jnp.float32)
            + 1.0) * np.pi
    rbf = np.float32((2.0 / _CUTOFF) ** 0.5) * jnp.sin(freq * (dist / _CUTOFF)) * dinv

    normed = diffs * dinv
    D = lax.dot_general(normed, normed, (((1,), (1,)), ((), ())),
                        preferred_element_type=jnp.float32)
    diag = jnp.sum(normed * normed, axis=1, keepdims=True)

    freqs2 = (lax.broadcasted_iota(jnp.int32, (1, _NSPH * _NRAD), 1).astype(jnp.float32)
              + 1.0) * np.pi
    rbf2 = jnp.sin(freqs2 * (dist / _CUTOFF)) * (_CUTOFF * dinv)

    xn = _mm(ch, p["embed"])
    xni = _mm(gi, xn)
    xnj = _mm(gj, xn)
    rbf_e = _act(_dense(p["edge_rbf"], rbf))
    wm = p["edge_mix"]["W"]
    x = _act(_mm(xni, wm[0:_EMB]) + _mm(xnj, wm[_EMB:2 * _EMB])
             + _mm(rbf_e, wm[2 * _EMB:3 * _EMB]) + p["edge_mix"]["b"])

    xs = [x]
    for b in p["blocks"]:
        x_ji = _act(_dense(b["ji"], x))
        x_kj = _act(_dense(b["kj"], x))
        rbf_p = _mm(_mm(rbf, b["rbf1"]["W"]), b["rbf2"]["W"])
        x_kj = _act(_dense(b["down"], x_kj * rbf_p))

        wsb = _mm(b["sbf1"]["W"], b["sbf2"]["W"])
        res = None
        self_t = None
        dk_mat = None
        dk_diag = None
        for k in range(_NSPH):
            wk = cmt[:, k:k + 1]
            kk = x_kj * _mm(rbf2, wsb * wk)
            if k == 0:
                res = jnp.sum(kk, axis=0, keepdims=True)
                self_t = kk
                dk_mat = D
                dk_diag = diag
            else:
                res = res + _mm(dk_mat, kk)
                self_t = self_t + dk_diag * kk
                if k < _NSPH - 1:
                    dk_mat = dk_mat * D
                    dk_diag = dk_diag * diag
        x_kj = (res - self_t) * np.float32(1.0 / (_E - 1))

        x_kj = _act(_dense(b["up"], x_kj))
        h = x_ji + x_kj
        for r in b["before"]:
            h = h + _act(_dense(r[1], _act(_dense(r[0], h))))
        h = x + _act(_dense(b["skip"], h))
        for r in b["after"]:
            h = h + _act(_dense(r[1], _act(_dense(r[0], h))))
        x = h
        xs.append(x)

    node_out = jnp.zeros((_N, 1), jnp.float32)
    for xi, o in zip(xs, p["outs"]):
        rbf_n = _mm(rbf, o["agg_rbf"]["W"])
        xw = rbf_n * xi
        node_x = _mm(git, xw) * np.float32(1.0 / (_N - 1))
        h = _mm(node_x, o["agg_out"]["W"])
        for mp in o["mlp"]:
            h = _act(_dense(mp, h))
        node_out = node_out + _mm(h, o["final"]["W"]) + _mm(ch, o["charge_bias"])
    out_ref[...] = node_out


def kernel(nuclei, params):
    flat, treedef = jax.tree_util.tree_flatten(params)
    flat = [jnp.reshape(a, (1, -1)) if a.ndim == 1 else a for a in flat]
    body = functools.partial(_body, treedef=treedef, n_flat=len(flat))
    out = pl.pallas_call(
        body,
        out_shape=jax.ShapeDtypeStruct((_N, 1), jnp.float32),
    )(nuclei, _GI, _GJ, _GIT, _CH, _CMT, *flat)
    return ([out], [])
